# Initial kernel scaffold; baseline (speedup 1.0000x reference)
import jax
import jax.numpy as jnp
from jax.experimental import pallas as pl


def kernel(x, edge_index, W1, b1, W2, b2, Wg, att_src, att_dst, bg, Wc, bc):
    raise NotImplementedError("write your pallas kernel here")


# speedup vs baseline: 14.2586x; 14.2586x over previous
import jax
import jax.numpy as jnp
from jax import lax
from jax.experimental import pallas as pl
from jax.experimental.pallas import tpu as pltpu
from jax.experimental.pallas import tpu_sc as plsc

N = 10000
E = 320000
D = 128
H = 64
C = 40
NP = 10240
NC, NS, L = 2, 16, 16
NW = NC * NS
EPW = E // NW
K = 80
NSTEP = EPW // K
RPT = NP // NS
GC = D + 16

_f32 = jnp.float32
_mesh = plsc.VectorSubcoreMesh(core_axis_name="c", subcore_axis_name="s")
_CP = pltpu.CompilerParams(needs_layout_passes=False, use_tc_tiling_on_sc=False)


def _zero_rows(rows, cols):
    for r in range(K):
        for c in range(cols // L):
            rows[r, pl.ds(c * L, L)] = jnp.zeros((L,), _f32)


def _fill_acc(rows, acc, sid):
    def cp(i, _):
        pltpu.sync_copy(rows, acc.at[pl.ds(sid * RPT + i * K, K)])
        return 0

    lax.fori_loop(0, RPT // K, cp, 0)


def _writeout(acc, rows, out, cid, sid):
    def wo(i, _):
        off = sid * RPT + i * K
        pltpu.sync_copy(acc.at[pl.ds(off, K)], rows)
        pltpu.sync_copy(rows, out.at[cid].at[pl.ds(off, K)])
        return 0

    lax.fori_loop(0, RPT // K, wo, 0)


def _make_sc_spmm(cols):
    def body(tab, src, dst, out, sidx, didx, rows, acc, sem):
        cid = lax.axis_index("c")
        sid = lax.axis_index("s")
        _zero_rows(rows, cols)
        _fill_acc(rows, acc, sid)
        plsc.subcore_barrier()
        base = (cid * NS + sid) * EPW

        def step(j, _):
            e0 = base + j * K
            pltpu.sync_copy(src.at[pl.ds(e0, K)], sidx)
            pltpu.sync_copy(dst.at[pl.ds(e0, K)], didx)
            pltpu.async_copy(tab.at[sidx], rows, sem).wait()
            pltpu.sync_copy(rows, acc.at[didx], add=True)
            return 0

        lax.fori_loop(0, NSTEP, step, 0)
        plsc.subcore_barrier()
        _writeout(acc, rows, out, cid, sid)

    return pl.kernel(
        body,
        out_type=jax.ShapeDtypeStruct((NC, NP, cols), _f32),
        mesh=_mesh,
        compiler_params=_CP,
        scratch_types=[
            pltpu.VMEM((K,), jnp.int32),
            pltpu.VMEM((K,), jnp.int32),
            pltpu.VMEM((K, cols), _f32),
            pltpu.VMEM_SHARED((NP, cols), _f32),
            pltpu.SemaphoreType.DMA,
        ],
    )


def _sc_gat_body(src, dst, asm, ad, mv, xaug, out,
                 sidx, didx, rows, tb, asm_v, ad_v, mvv, acc, sem):
    cid = lax.axis_index("c")
    sid = lax.axis_index("s")
    _zero_rows(rows, GC)
    _fill_acc(rows, acc, sid)
    pltpu.sync_copy(asm, asm_v)
    pltpu.sync_copy(ad, ad_v)
    pltpu.sync_copy(mv, mvv)
    plsc.subcore_barrier()
    base = (cid * NS + sid) * EPW

    def step(j, _):
        e0 = base + j * K
        pltpu.sync_copy(src.at[pl.ds(e0, K)], sidx)
        pltpu.sync_copy(dst.at[pl.ds(e0, K)], didx)
        cp = pltpu.async_copy(xaug.at[sidx], rows, sem)
        m = mvv[...]

        def tblk(i, _):
            sv = sidx[pl.ds(i * L, L)]
            dv = didx[pl.ds(i * L, L)]
            v = plsc.load_gather(asm_v, [sv]) + plsc.load_gather(ad_v, [dv])
            tb[pl.ds(i * L, L)] = jnp.exp(jnp.maximum(v, 0.2 * v) - m)
            return 0

        lax.fori_loop(0, K // L, tblk, 0)
        cp.wait()

        def rowm(r, _):
            t = plsc.load_gather(tb, [jnp.full((L,), r, jnp.int32)])
            for c in range(GC // L):
                rows[r, pl.ds(c * L, L)] = rows[r, pl.ds(c * L, L)] * t
            return 0

        lax.fori_loop(0, K, rowm, 0)
        pltpu.sync_copy(rows, acc.at[didx], add=True)
        return 0

    lax.fori_loop(0, NSTEP, step, 0)
    plsc.subcore_barrier()
    _writeout(acc, rows, out, cid, sid)


_sc_gat = pl.kernel(
    _sc_gat_body,
    out_type=jax.ShapeDtypeStruct((NC, NP, GC), _f32),
    mesh=_mesh,
    compiler_params=_CP,
    scratch_types=[
        pltpu.VMEM((K,), jnp.int32),
        pltpu.VMEM((K,), jnp.int32),
        pltpu.VMEM((K, GC), _f32),
        pltpu.VMEM((K,), _f32),
        pltpu.VMEM((NP,), _f32),
        pltpu.VMEM((NP,), _f32),
        pltpu.VMEM((L,), _f32),
        pltpu.VMEM_SHARED((NP, GC), _f32),
        pltpu.SemaphoreType.DMA,
    ],
)


_BLK = 1024
_GRID = NP // _BLK


def _rows_spec(cols):
    return pl.BlockSpec((_BLK, cols), lambda i: (i, 0))


def _full_spec(r, c):
    return pl.BlockSpec((r, c), lambda i: (0, 0))


def _tc1_body(x_ref, kf_ref, d0_ref, d1_ref, w1_ref,
              xl1_ref, u1_ref, dinv_ref):
    kf = kf_ref[...]
    xl1 = jnp.dot(kf * x_ref[...], w1_ref[...], preferred_element_type=_f32)
    deg = d0_ref[...][:, 0:1] + d1_ref[...][:, 0:1] + 1.0
    dinv = lax.rsqrt(deg)
    xl1_ref[...] = xl1
    u1_ref[...] = (kf * dinv) * xl1
    dinv_ref[...] = dinv


_tc1 = pl.pallas_call(
    _tc1_body,
    grid=(_GRID,),
    in_specs=[_rows_spec(D), _rows_spec(1), _rows_spec(16), _rows_spec(16),
              _full_spec(D, 2 * H)],
    out_specs=[_rows_spec(2 * H), _rows_spec(2 * H), _rows_spec(1)],
    out_shape=[jax.ShapeDtypeStruct((NP, 2 * H), _f32),
               jax.ShapeDtypeStruct((NP, 2 * H), _f32),
               jax.ShapeDtypeStruct((NP, 1), _f32)],
)


def _tc_ktab_body(kf_ref, ktab_ref):
    ktab_ref[...] = jnp.concatenate(
        [kf_ref[...], jnp.zeros((_BLK, 15), _f32)], axis=1)


_tc_ktab = pl.pallas_call(
    _tc_ktab_body,
    grid=(_GRID,),
    in_specs=[_rows_spec(1)],
    out_specs=_rows_spec(16),
    out_shape=jax.ShapeDtypeStruct((NP, 16), _f32),
)


def _tc2_body(a0_ref, a1_ref, xl1_ref, dinv_ref, kf_ref, b1_ref, w2_ref,
              xl2_ref, u2_ref):
    dinv = dinv_ref[...]
    kf = kf_ref[...]
    h1 = jax.nn.relu(dinv * (a0_ref[...] + a1_ref[...])
                     + (dinv * dinv) * xl1_ref[...] + b1_ref[...])
    xl2 = jnp.dot(h1, w2_ref[...], preferred_element_type=_f32)
    xl2_ref[...] = xl2
    u2_ref[...] = (kf * dinv) * xl2


_tc2 = pl.pallas_call(
    _tc2_body,
    grid=(_GRID,),
    in_specs=[_rows_spec(2 * H), _rows_spec(2 * H), _rows_spec(2 * H),
              _rows_spec(1), _rows_spec(1), _full_spec(1, 2 * H),
              _full_spec(2 * H, H)],
    out_specs=[_rows_spec(H), _rows_spec(H)],
    out_shape=[jax.ShapeDtypeStruct((NP, H), _f32),
               jax.ShapeDtypeStruct((NP, H), _f32)],
)


def _tc3_body(a0_ref, a1_ref, xl2_ref, dinv_ref, kf_ref, b2_ref, wg_ref,
              asrc_ref, adst_ref,
              xaug_ref, asm_ref, ad_ref, ma_ref, md_ref):
    i = pl.program_id(0)
    dinv = dinv_ref[...]
    kf = kf_ref[...]
    h2 = jax.nn.relu(dinv * (a0_ref[...] + a1_ref[...])
                     + (dinv * dinv) * xl2_ref[...] + b2_ref[...])
    xl3 = jnp.dot(h2, wg_ref[...], preferred_element_type=_f32)
    as_ = jnp.dot(xl3, asrc_ref[...], preferred_element_type=_f32)
    ad_ = jnp.dot(xl3, adst_ref[...], preferred_element_type=_f32)
    asm = jnp.where(kf > 0, as_, -1e30)
    xaug_ref[...] = jnp.concatenate(
        [xl3, jnp.ones((_BLK, 1), _f32), jnp.zeros((_BLK, 15), _f32)], axis=1)
    asm_ref[...] = asm
    ad_ref[...] = ad_

    @pl.when(i == 0)
    def _():
        ma_ref[...] = jnp.full((1, 1), -1e30, _f32)
        md_ref[...] = jnp.full((1, 1), -1e30, _f32)

    ma_ref[...] = jnp.maximum(ma_ref[...], jnp.max(asm))
    md_ref[...] = jnp.maximum(md_ref[...], jnp.max(ad_))


_tc3 = pl.pallas_call(
    _tc3_body,
    grid=(_GRID,),
    in_specs=[_rows_spec(H), _rows_spec(H), _rows_spec(H),
              _rows_spec(1), _rows_spec(1), _full_spec(1, H),
              _full_spec(H, D), _full_spec(D, 1), _full_spec(D, 1)],
    out_specs=[_rows_spec(GC), _rows_spec(1), _rows_spec(1),
               _full_spec(1, 1), _full_spec(1, 1)],
    out_shape=[jax.ShapeDtypeStruct((NP, GC), _f32),
               jax.ShapeDtypeStruct((NP, 1), _f32),
               jax.ShapeDtypeStruct((NP, 1), _f32),
               jax.ShapeDtypeStruct((1, 1), _f32),
               jax.ShapeDtypeStruct((1, 1), _f32)],
)


def _tc4_body(g0_ref, g1_ref, bg_ref, wc_ref, bc_ref, out_ref):
    g = g0_ref[...] + g1_ref[...]
    s = g[:, D:D + 1]
    z = jax.nn.relu(g[:, :D] / (s + 1e-16) + bg_ref[...])
    xr = 1.0 / (1.0 + jnp.exp(-z))
    lg = jnp.dot(xr, wc_ref[...], preferred_element_type=_f32) + bc_ref[...]
    m = jnp.max(lg, axis=1, keepdims=True)
    e = lg - m
    out_ref[...] = e - jnp.log(jnp.sum(jnp.exp(e), axis=1, keepdims=True))


_tc4 = pl.pallas_call(
    _tc4_body,
    grid=(_GRID,),
    in_specs=[_rows_spec(GC), _rows_spec(GC), _full_spec(1, D),
              _full_spec(D, C), _full_spec(1, C)],
    out_specs=_rows_spec(C),
    out_shape=jax.ShapeDtypeStruct((NP, C), _f32),
)

_sc_spmm_deg = _make_sc_spmm(16)
_sc_spmm_128 = _make_sc_spmm(2 * H)
_sc_spmm_64 = _make_sc_spmm(H)


def kernel(x, edge_index, W1, b1, W2, b2, Wg, att_src, att_dst, bg, Wc, bc):
    perm = jax.random.permutation(jax.random.key(42), N)
    mask_nodes = perm[: int(0.15 * N)]
    keptf = jnp.ones((N,), _f32).at[mask_nodes].set(0.0)
    kf_p = jnp.zeros((NP, 1), _f32).at[:N, 0].set(keptf)
    x_p = jnp.zeros((NP, D), _f32).at[:N].set(x)
    src = edge_index[0]
    dst = edge_index[1]

    ktab = _tc_ktab(kf_p)
    degp = _sc_spmm_deg(ktab, src, dst)
    xl1, u1, dinv = _tc1(x_p, kf_p, degp[0], degp[1], W1)
    agg1 = _sc_spmm_128(u1, src, dst)
    xl2, u2 = _tc2(agg1[0], agg1[1], xl1, dinv, kf_p,
                   b1.reshape(1, 2 * H), W2)
    agg2 = _sc_spmm_64(u2, src, dst)
    xaug, asm, ad, ma, md = _tc3(agg2[0], agg2[1], xl2, dinv, kf_p,
                                 b2.reshape(1, H), Wg,
                                 att_src.reshape(D, 1), att_dst.reshape(D, 1))
    mglob = jnp.maximum(ma[0, 0] + md[0, 0], 0.0)
    mvec = jnp.full((L,), mglob, _f32)
    gat = _sc_gat(src, dst, asm.reshape(NP), ad.reshape(NP), mvec, xaug)
    out = _tc4(gat[0], gat[1], bg.reshape(1, D), Wc, bc.reshape(1, C))
    return out[:N]
